# baseline (speedup 1.0000x reference)
import functools

import jax
import jax.numpy as jnp
from jax import lax
from jax.experimental import pallas as pl
from jax.experimental.pallas import tpu as pltpu
from jax.experimental.pallas import tpu_sc as plsc

NC = 2
NS = 16
NW = NC * NS


def _wid():
    return lax.axis_index("s") * NC + lax.axis_index("c")


_SC_PARAMS = pltpu.CompilerParams(needs_layout_passes=False,
                                  use_tc_tiling_on_sc=False)


def _hist_call(cols, zeros_n):
    (E,) = cols.shape
    (N,) = zeros_n.shape
    ep = E // NW
    mesh = plsc.VectorSubcoreMesh(core_axis_name="c", subcore_axis_name="s")

    @functools.partial(
        pl.kernel, mesh=mesh, compiler_params=_SC_PARAMS,
        out_type=jax.ShapeDtypeStruct((NW, N), jnp.float32),
        scratch_types=[
            pltpu.VMEM((ep,), jnp.int32),
            pltpu.VMEM((N,), jnp.float32),
        ],
    )
    def k(cols_hbm, zeros_hbm, out_hbm, cidx_v, hist_v):
        w = _wid()
        pltpu.sync_copy(cols_hbm.at[pl.ds(w * ep, ep)], cidx_v)
        pltpu.sync_copy(zeros_hbm, hist_v)
        ones = jnp.ones((16,), jnp.float32)

        def body(i, c):
            idx = cidx_v[pl.ds(i * 16, 16)]
            plsc.addupdate_scatter(hist_v, [idx], ones)
            return c

        lax.fori_loop(0, ep // 16, body, 0, unroll=4)
        pltpu.sync_copy(hist_v, out_hbm.at[w])

    return k(cols, zeros_n)


def _scatter_call(rows, cols, y, zeros_nh, C):
    (E,) = rows.shape
    N, H = y.shape
    ep = E // NW
    CH = ep // C
    NB = 5
    mesh = plsc.VectorSubcoreMesh(core_axis_name="c", subcore_axis_name="s")

    @functools.partial(
        pl.kernel, mesh=mesh, compiler_params=_SC_PARAMS,
        out_type=jax.ShapeDtypeStruct((NC, N, H), jnp.float32),
        scratch_types=[
            pltpu.VMEM((ep,), jnp.int32),
            pltpu.VMEM((ep,), jnp.int32),
            pltpu.VMEM((NB, C, H), jnp.float32),
            pltpu.VMEM_SHARED((N, H), jnp.float32),
        ] + [pltpu.SemaphoreType.DMA] * NB,
    )
    def k(rows_hbm, cols_hbm, y_hbm, zeros_hbm, out_hbm,
          ridx_v, cidx_v, yg_v, acc_sh, *sems):
        cid = lax.axis_index("c")
        sid = lax.axis_index("s")
        w = sid * NC + cid
        pltpu.sync_copy(rows_hbm.at[pl.ds(w * ep, ep)], ridx_v)
        pltpu.sync_copy(cols_hbm.at[pl.ds(w * ep, ep)], cidx_v)

        @pl.when(sid == 0)
        def _():
            pltpu.sync_copy(zeros_hbm, acc_sh)

        plsc.subcore_barrier()

        def start(j, b):
            pltpu.async_copy(
                y_hbm.at[ridx_v.at[pl.ds(j * C, C)]], yg_v.at[b], sems[b])

        for b in range(NB):
            start(b, b)

        @pl.loop(0, CH, step=NB)
        def _outer(i):
            for b in range(NB):
                j = i + b
                pltpu.make_async_copy(
                    y_hbm.at[ridx_v.at[pl.ds(j * C, C)]], yg_v.at[b],
                    sems[b]).wait()
                pltpu.sync_copy(
                    yg_v.at[b], acc_sh.at[cidx_v.at[pl.ds(j * C, C)]],
                    add=True)

                @pl.when(j + NB < CH)
                def _():
                    start(j + NB, b)

        plsc.subcore_barrier()

        @pl.when(sid == 0)
        def _():
            pltpu.sync_copy(acc_sh, out_hbm.at[cid])

    return k(rows, cols, y, zeros_nh)


def _decoder_call(rows, cols, A2, B2, sgn, nl, b16, C):
    (E,) = rows.shape
    N, K = A2.shape
    ep = E // NW
    CH = ep // C
    G = C // 16
    NB = 5
    mesh = plsc.VectorSubcoreMesh(core_axis_name="c", subcore_axis_name="s")

    @functools.partial(
        pl.kernel, mesh=mesh, compiler_params=_SC_PARAMS,
        out_type=jax.ShapeDtypeStruct((E,), jnp.float32),
        scratch_types=[
            pltpu.VMEM((ep,), jnp.int32),
            pltpu.VMEM((ep,), jnp.int32),
            pltpu.VMEM((NB, C, K), jnp.float32),
            pltpu.VMEM((NB, C, K), jnp.float32),
            pltpu.VMEM((K,), jnp.float32),
            pltpu.VMEM((ep,), jnp.float32),
            pltpu.VMEM((ep,), jnp.float32),
            pltpu.VMEM((16,), jnp.float32),
        ] + [pltpu.SemaphoreType.DMA] * (2 * NB),
    )
    def k(rows_hbm, cols_hbm, a_hbm, b_hbm, sgn_hbm, nl_hbm, b16_hbm,
          out_hbm, ridx_v, cidx_v, ar_v, bc_v, sgn_v, nl_v, ob_v, b16_v,
          *sems):
        sems_a = sems[:NB]
        sems_b = sems[NB:]
        w = _wid()
        pltpu.sync_copy(rows_hbm.at[pl.ds(w * ep, ep)], ridx_v)
        pltpu.sync_copy(cols_hbm.at[pl.ds(w * ep, ep)], cidx_v)
        pltpu.sync_copy(nl_hbm.at[pl.ds(w * ep, ep)], nl_v)
        pltpu.sync_copy(sgn_hbm, sgn_v)
        pltpu.sync_copy(b16_hbm, b16_v)
        bval = b16_v[...]
        lanes = jnp.arange(16, dtype=jnp.int32)

        def start(j, b):
            pltpu.async_copy(a_hbm.at[ridx_v.at[pl.ds(j * C, C)]],
                             ar_v.at[b], sems_a[b])
            pltpu.async_copy(b_hbm.at[cidx_v.at[pl.ds(j * C, C)]],
                             bc_v.at[b], sems_b[b])

        for b in range(NB):
            start(b, b)

        @pl.loop(0, CH, step=NB)
        def _outer(i):
            for b in range(NB):
                j = i + b
                pltpu.make_async_copy(
                    a_hbm.at[ridx_v.at[pl.ds(j * C, C)]], ar_v.at[b],
                    sems_a[b]).wait()
                pltpu.make_async_copy(
                    b_hbm.at[cidx_v.at[pl.ds(j * C, C)]], bc_v.at[b],
                    sems_b[b]).wait()
                base = j * C
                bsp = jnp.zeros((16,), jnp.int32) + b

                def kbody(kk, accs):
                    kidx = (lanes + kk) & (K - 1)
                    sgn_r = plsc.load_gather(sgn_v, [kidx])
                    out = []
                    for g in range(G):
                        r = lanes + (g * 16)
                        a = plsc.load_gather(ar_v, [bsp, r, kidx])
                        bb = plsc.load_gather(bc_v, [bsp, r, kidx])
                        u = jnp.maximum(a + bb, 0.0)
                        out.append(accs[g] + u * sgn_r)
                    return tuple(out)

                accs = lax.fori_loop(
                    0, K, kbody,
                    tuple(jnp.zeros((16,), jnp.float32) for _ in range(G)),
                    unroll=4)
                for g in range(G):
                    o = accs[g] + bval + nl_v[pl.ds(base + g * 16, 16)]
                    ob_v[pl.ds(base + g * 16, 16)] = \
                        1.0 / (1.0 + jnp.exp(-o))

                @pl.when(j + NB < CH)
                def _():
                    start(j + NB, b)

        pltpu.sync_copy(ob_v, out_hbm.at[pl.ds(w * ep, ep)])

    return k(rows, cols, A2, B2, sgn, nl, b16)


HP = 24


def _extract_call(edge_index):
    E = edge_index.shape[1]
    R = E // 128

    def body(ei_ref, rows_ref, cols_ref):
        rows_ref[...] = ei_ref[0, :].reshape(R, 128)
        cols_ref[...] = ei_ref[1, :].reshape(R, 128)

    rows2, cols2 = pl.pallas_call(
        body,
        out_shape=(jax.ShapeDtypeStruct((R, 128), jnp.int32),
                   jax.ShapeDtypeStruct((R, 128), jnp.int32)),
    )(edge_index)
    return rows2.reshape(E), cols2.reshape(E)


def _tc_pre_call(x, W_gcn, hist_t):
    N, D = x.shape
    H = W_gcn.shape[1]

    def body(x_ref, w_ref, h_ref, y_ref, dis_ref):
        deg_l = jnp.sum(h_ref[...], axis=0, keepdims=True) + 1.0
        deg = jnp.transpose(deg_l)
        dis = lax.rsqrt(deg)
        xw = jnp.dot(x_ref[...], w_ref[...],
                     preferred_element_type=jnp.float32)
        yv = xw * dis
        y_ref[...] = jnp.concatenate(
            [yv, jnp.zeros((N, HP - H), jnp.float32)], axis=1)
        dis_ref[...] = dis

    return pl.pallas_call(
        body,
        out_shape=(jax.ShapeDtypeStruct((N, HP), jnp.float32),
                   jax.ShapeDtypeStruct((N, 1), jnp.float32)),
    )(x, W_gcn, hist_t)


def _tc_tables_call(y, dis, acc_parts, bg, W1, b1, w2r, nid):
    N = y.shape[0]
    H = bg.shape[1]
    K = W1.shape[1]

    def body(y_ref, dis_ref, acc_ref, bg_ref, w1_ref, b1_ref, w2_ref,
             nid_ref, a_ref, b_ref, sgn_ref, enc_ref):
        acc = acc_ref[0, :, 0:H] + acc_ref[1, :, 0:H]
        enc = jnp.maximum(
            dis_ref[...] * (acc + y_ref[:, 0:H]) + bg_ref[...], 0.0)
        enc_ref[...] = enc
        nid = nid_ref[0]
        erow = enc_ref[pl.ds(nid, 1), :]
        w1a = w1_ref[0:H, :]
        w1b = w1_ref[H:2 * H, :]
        w1c = w1_ref[2 * H:3 * H, :]
        cvec = jnp.dot(erow, w1c, preferred_element_type=jnp.float32) \
            + b1_ref[...]
        aw2 = jnp.abs(w2_ref[...])
        a_ref[...] = (jnp.dot(enc, w1a, preferred_element_type=jnp.float32)
                      + cvec) * aw2
        b_ref[...] = jnp.dot(enc, w1b,
                             preferred_element_type=jnp.float32) * aw2
        sgn_ref[...] = jnp.sign(w2_ref[...])

    vm = pl.BlockSpec(memory_space=pltpu.VMEM)
    return pl.pallas_call(
        body,
        in_specs=[vm, vm, vm, vm, vm, vm, vm,
                  pl.BlockSpec(memory_space=pltpu.SMEM)],
        out_specs=(vm, vm, vm),
        out_shape=(jax.ShapeDtypeStruct((N, K), jnp.float32),
                   jax.ShapeDtypeStruct((N, K), jnp.float32),
                   jax.ShapeDtypeStruct((1, K), jnp.float32)),
        scratch_shapes=[pltpu.VMEM((N, H), jnp.float32)],
    )(y, dis, acc_parts, bg, W1, b1, w2r, nid)


def kernel(x, edge_index, node_id, W_gcn, b_gcn, W1, b1, W2, b2):
    N, D = x.shape
    H = W_gcn.shape[1]
    E = edge_index.shape[1]
    K = W1.shape[1]

    rows, cols = _extract_call(edge_index)

    bias = 0.0 + 0.0001
    eps = (bias - (1.0 - bias)) * jax.random.uniform(
        jax.random.key(42), (E,), dtype=jnp.float32) + (1.0 - bias)
    nl = jnp.log(eps) - jnp.log(1.0 - eps)
    b16 = jnp.broadcast_to(b2, (16,))

    zeros_n = jnp.zeros((N,), jnp.float32)
    zeros_nh = jnp.zeros((N, HP), jnp.float32)

    C = 80

    hist_parts = _hist_call(cols, zeros_n)
    y, dis = _tc_pre_call(x, W_gcn, hist_parts)
    acc_parts = _scatter_call(rows, cols, y, zeros_nh, C)
    a2, b2t, sgn = _tc_tables_call(
        y, dis, acc_parts, b_gcn.reshape(1, H), W1, b1.reshape(1, K),
        W2.reshape(1, K), jnp.asarray(node_id, jnp.int32).reshape(1))
    out = _decoder_call(rows, cols, a2, b2t, sgn.reshape(K), nl, b16, C)
    return out.reshape(E, 1)


# speedup vs baseline: 1.0041x; 1.0041x over previous
import functools

import jax
import jax.numpy as jnp
from jax import lax
from jax.experimental import pallas as pl
from jax.experimental.pallas import tpu as pltpu
from jax.experimental.pallas import tpu_sc as plsc

NC = 2
NS = 16
NW = NC * NS


def _wid():
    return lax.axis_index("s") * NC + lax.axis_index("c")


_SC_PARAMS = pltpu.CompilerParams(needs_layout_passes=False,
                                  use_tc_tiling_on_sc=False)


def _hist_call(cols, zeros_n):
    (E,) = cols.shape
    (N,) = zeros_n.shape
    ep = E // NW
    mesh = plsc.VectorSubcoreMesh(core_axis_name="c", subcore_axis_name="s")

    @functools.partial(
        pl.kernel, mesh=mesh, compiler_params=_SC_PARAMS,
        out_type=jax.ShapeDtypeStruct((NW, N), jnp.float32),
        scratch_types=[
            pltpu.VMEM((ep,), jnp.int32),
            pltpu.VMEM((N,), jnp.float32),
        ],
    )
    def k(cols_hbm, zeros_hbm, out_hbm, cidx_v, hist_v):
        w = _wid()
        pltpu.sync_copy(cols_hbm.at[pl.ds(w * ep, ep)], cidx_v)
        pltpu.sync_copy(zeros_hbm, hist_v)
        ones = jnp.ones((16,), jnp.float32)

        def body(i, c):
            idx = cidx_v[pl.ds(i * 16, 16)]
            plsc.addupdate_scatter(hist_v, [idx], ones)
            return c

        lax.fori_loop(0, ep // 16, body, 0, unroll=4)
        pltpu.sync_copy(hist_v, out_hbm.at[w])

    return k(cols, zeros_n)


def _scatter_call(rows, cols, y, zeros_nh, C):
    (E,) = rows.shape
    N, H = y.shape
    ep = E // NW
    CH = ep // C
    NB = 5
    mesh = plsc.VectorSubcoreMesh(core_axis_name="c", subcore_axis_name="s")

    @functools.partial(
        pl.kernel, mesh=mesh, compiler_params=_SC_PARAMS,
        out_type=jax.ShapeDtypeStruct((NC, N, H), jnp.float32),
        scratch_types=[
            pltpu.VMEM((ep,), jnp.int32),
            pltpu.VMEM((ep,), jnp.int32),
            pltpu.VMEM((NB, C, H), jnp.float32),
            pltpu.VMEM_SHARED((N, H), jnp.float32),
        ] + [pltpu.SemaphoreType.DMA] * NB,
    )
    def k(rows_hbm, cols_hbm, y_hbm, zeros_hbm, out_hbm,
          ridx_v, cidx_v, yg_v, acc_sh, *sems):
        cid = lax.axis_index("c")
        sid = lax.axis_index("s")
        w = sid * NC + cid
        pltpu.sync_copy(rows_hbm.at[pl.ds(w * ep, ep)], ridx_v)
        pltpu.sync_copy(cols_hbm.at[pl.ds(w * ep, ep)], cidx_v)

        @pl.when(sid == 0)
        def _():
            pltpu.sync_copy(zeros_hbm, acc_sh)

        plsc.subcore_barrier()

        def start(j, b):
            pltpu.async_copy(
                y_hbm.at[ridx_v.at[pl.ds(j * C, C)]], yg_v.at[b], sems[b])

        for b in range(NB):
            start(b, b)

        @pl.loop(0, CH, step=NB)
        def _outer(i):
            for b in range(NB):
                j = i + b
                pltpu.make_async_copy(
                    y_hbm.at[ridx_v.at[pl.ds(j * C, C)]], yg_v.at[b],
                    sems[b]).wait()
                pltpu.sync_copy(
                    yg_v.at[b], acc_sh.at[cidx_v.at[pl.ds(j * C, C)]],
                    add=True)

                @pl.when(j + NB < CH)
                def _():
                    start(j + NB, b)

        plsc.subcore_barrier()

        @pl.when(sid == 0)
        def _():
            pltpu.sync_copy(acc_sh, out_hbm.at[cid])

    return k(rows, cols, y, zeros_nh)


def _decoder_call(rows, cols, A2, B2, sgn, nl, b16, C):
    (E,) = rows.shape
    N, K = A2.shape
    ep = E // NW
    CH = ep // C
    G = C // 16
    NB = 5
    mesh = plsc.VectorSubcoreMesh(core_axis_name="c", subcore_axis_name="s")

    @functools.partial(
        pl.kernel, mesh=mesh, compiler_params=_SC_PARAMS,
        out_type=jax.ShapeDtypeStruct((E,), jnp.float32),
        scratch_types=[
            pltpu.VMEM((ep,), jnp.int32),
            pltpu.VMEM((ep,), jnp.int32),
            pltpu.VMEM((NB, C, K), jnp.float32),
            pltpu.VMEM((NB, C, K), jnp.float32),
            pltpu.VMEM((K,), jnp.float32),
            pltpu.VMEM((ep,), jnp.float32),
            pltpu.VMEM((ep,), jnp.float32),
            pltpu.VMEM((16,), jnp.float32),
        ] + [pltpu.SemaphoreType.DMA] * (2 * NB),
    )
    def k(rows_hbm, cols_hbm, a_hbm, b_hbm, sgn_hbm, nl_hbm, b16_hbm,
          out_hbm, ridx_v, cidx_v, ar_v, bc_v, sgn_v, nl_v, ob_v, b16_v,
          *sems):
        sems_a = sems[:NB]
        sems_b = sems[NB:]
        w = _wid()
        pltpu.sync_copy(rows_hbm.at[pl.ds(w * ep, ep)], ridx_v)
        pltpu.sync_copy(cols_hbm.at[pl.ds(w * ep, ep)], cidx_v)
        pltpu.sync_copy(nl_hbm.at[pl.ds(w * ep, ep)], nl_v)
        pltpu.sync_copy(sgn_hbm, sgn_v)
        pltpu.sync_copy(b16_hbm, b16_v)
        bval = b16_v[...]
        lanes = jnp.arange(16, dtype=jnp.int32)

        def start(j, b):
            pltpu.async_copy(a_hbm.at[ridx_v.at[pl.ds(j * C, C)]],
                             ar_v.at[b], sems_a[b])
            pltpu.async_copy(b_hbm.at[cidx_v.at[pl.ds(j * C, C)]],
                             bc_v.at[b], sems_b[b])

        for b in range(NB):
            start(b, b)

        @pl.loop(0, CH, step=NB)
        def _outer(i):
            for b in range(NB):
                j = i + b
                pltpu.make_async_copy(
                    a_hbm.at[ridx_v.at[pl.ds(j * C, C)]], ar_v.at[b],
                    sems_a[b]).wait()
                pltpu.make_async_copy(
                    b_hbm.at[cidx_v.at[pl.ds(j * C, C)]], bc_v.at[b],
                    sems_b[b]).wait()
                base = j * C
                bsp = jnp.zeros((16,), jnp.int32) + b

                def kbody(kk, accs):
                    kidx = (lanes + kk) & (K - 1)
                    sgn_r = plsc.load_gather(sgn_v, [kidx])
                    out = []
                    for g in range(G):
                        r = lanes + (g * 16)
                        a = plsc.load_gather(ar_v, [bsp, r, kidx])
                        bb = plsc.load_gather(bc_v, [bsp, r, kidx])
                        u = jnp.maximum(a + bb, 0.0)
                        out.append(accs[g] + u * sgn_r)
                    return tuple(out)

                accs = lax.fori_loop(
                    0, K, kbody,
                    tuple(jnp.zeros((16,), jnp.float32) for _ in range(G)),
                    unroll=4)
                for g in range(G):
                    o = accs[g] + bval + nl_v[pl.ds(base + g * 16, 16)]
                    ob_v[pl.ds(base + g * 16, 16)] = \
                        1.0 / (1.0 + jnp.exp(-o))

                @pl.when(j + NB < CH)
                def _():
                    start(j + NB, b)

        pltpu.sync_copy(ob_v, out_hbm.at[pl.ds(w * ep, ep)])

    return k(rows, cols, A2, B2, sgn, nl, b16)


HP = 32


def _extract_call(edge_index):
    E = edge_index.shape[1]
    R = E // 128

    def body(ei_ref, rows_ref, cols_ref):
        rows_ref[...] = ei_ref[0, :].reshape(R, 128)
        cols_ref[...] = ei_ref[1, :].reshape(R, 128)

    rows2, cols2 = pl.pallas_call(
        body,
        out_shape=(jax.ShapeDtypeStruct((R, 128), jnp.int32),
                   jax.ShapeDtypeStruct((R, 128), jnp.int32)),
    )(edge_index)
    return rows2.reshape(E), cols2.reshape(E)


def _tc_pre_call(x, W_gcn, hist_t):
    N, D = x.shape
    H = W_gcn.shape[1]

    def body(x_ref, w_ref, h_ref, y_ref, dis_ref):
        deg_l = jnp.sum(h_ref[...], axis=0, keepdims=True) + 1.0
        deg = jnp.transpose(deg_l)
        dis = lax.rsqrt(deg)
        xw = jnp.dot(x_ref[...], w_ref[...],
                     preferred_element_type=jnp.float32)
        yv = xw * dis
        y_ref[...] = jnp.concatenate(
            [yv, jnp.zeros((N, HP - H), jnp.float32)], axis=1)
        dis_ref[...] = dis

    return pl.pallas_call(
        body,
        out_shape=(jax.ShapeDtypeStruct((N, HP), jnp.float32),
                   jax.ShapeDtypeStruct((N, 1), jnp.float32)),
    )(x, W_gcn, hist_t)


def _tc_tables_call(y, dis, acc_parts, bg, W1, b1, w2r, nid):
    N = y.shape[0]
    H = bg.shape[1]
    K = W1.shape[1]

    def body(y_ref, dis_ref, acc_ref, bg_ref, w1_ref, b1_ref, w2_ref,
             nid_ref, a_ref, b_ref, sgn_ref, enc_ref):
        acc = acc_ref[0, :, 0:H] + acc_ref[1, :, 0:H]
        enc = jnp.maximum(
            dis_ref[...] * (acc + y_ref[:, 0:H]) + bg_ref[...], 0.0)
        enc_ref[...] = enc
        nid = nid_ref[0]
        erow = enc_ref[pl.ds(nid, 1), :]
        w1a = w1_ref[0:H, :]
        w1b = w1_ref[H:2 * H, :]
        w1c = w1_ref[2 * H:3 * H, :]
        cvec = jnp.dot(erow, w1c, preferred_element_type=jnp.float32) \
            + b1_ref[...]
        aw2 = jnp.abs(w2_ref[...])
        a_ref[...] = (jnp.dot(enc, w1a, preferred_element_type=jnp.float32)
                      + cvec) * aw2
        b_ref[...] = jnp.dot(enc, w1b,
                             preferred_element_type=jnp.float32) * aw2
        sgn_ref[...] = jnp.sign(w2_ref[...])

    vm = pl.BlockSpec(memory_space=pltpu.VMEM)
    return pl.pallas_call(
        body,
        in_specs=[vm, vm, vm, vm, vm, vm, vm,
                  pl.BlockSpec(memory_space=pltpu.SMEM)],
        out_specs=(vm, vm, vm),
        out_shape=(jax.ShapeDtypeStruct((N, K), jnp.float32),
                   jax.ShapeDtypeStruct((N, K), jnp.float32),
                   jax.ShapeDtypeStruct((1, K), jnp.float32)),
        scratch_shapes=[pltpu.VMEM((N, H), jnp.float32)],
    )(y, dis, acc_parts, bg, W1, b1, w2r, nid)


def kernel(x, edge_index, node_id, W_gcn, b_gcn, W1, b1, W2, b2):
    N, D = x.shape
    H = W_gcn.shape[1]
    E = edge_index.shape[1]
    K = W1.shape[1]

    rows, cols = _extract_call(edge_index)

    bias = 0.0 + 0.0001
    eps = (bias - (1.0 - bias)) * jax.random.uniform(
        jax.random.key(42), (E,), dtype=jnp.float32) + (1.0 - bias)
    nl = jnp.log(eps) - jnp.log(1.0 - eps)
    b16 = jnp.broadcast_to(b2, (16,))

    zeros_n = jnp.zeros((N,), jnp.float32)
    zeros_nh = jnp.zeros((N, HP), jnp.float32)

    C = 80

    hist_parts = _hist_call(cols, zeros_n)
    y, dis = _tc_pre_call(x, W_gcn, hist_parts)
    acc_parts = _scatter_call(rows, cols, y, zeros_nh, C)
    a2, b2t, sgn = _tc_tables_call(
        y, dis, acc_parts, b_gcn.reshape(1, H), W1, b1.reshape(1, K),
        W2.reshape(1, K), jnp.asarray(node_id, jnp.int32).reshape(1))
    out = _decoder_call(rows, cols, a2, b2t, sgn.reshape(K), nl, b16, C)
    return out.reshape(E, 1)
